# baseline (speedup 1.0000x reference)
import functools

import jax
import jax.numpy as jnp
from jax import lax
from jax.experimental import pallas as pl
from jax.experimental.pallas import tpu as pltpu
from jax.experimental.pallas import tpu_sc as plsc

VOCAB = 1000000
D_MODEL = 64
SCALE = 8.0

NC, NS = 2, 16
NW = NC * NS
LANES = 16

P1_W = 256
NSTRIP = VOCAB // P1_W
REM_BASE = NSTRIP * P1_W
TSROWS = VOCAB // 2
P1_NBUF = 2
P1_MAXK = (NSTRIP + NW - 1) // NW
P1_NT = (P1_MAXK + P1_NBUF - 1) // P1_NBUF

SEQ, TOK = 4096, 200
SBLK = 128
P2_NBUF = 4
P2_NOB = 2

_mesh = plsc.VectorSubcoreMesh(
    core_axis_name="c", subcore_axis_name="s", num_cores=NC, num_subcores=NS
)
_tc_tiled = pltpu.CompilerParams(
    use_tc_tiling_on_sc=True, needs_layout_passes=False
)


def _wid():
    return lax.axis_index("s") * NC + lax.axis_index("c")


def _transpose_strip(in_v, ob_v, nrow, riota):
    @plsc.parallel_loop(0, nrow, unroll=8)
    def row(r):
        for h in range(2):
            col = jnp.full((LANES,), 2 * r + h, jnp.int32)
            for j in range(4):
                v = plsc.load_gather(in_v, [riota[j], col])
                ob_v[r, pl.ds(h * 64 + 16 * j, LANES)] = v * SCALE


@functools.partial(
    pl.kernel,
    out_type=jax.ShapeDtypeStruct((TSROWS, 128), jnp.float32),
    mesh=_mesh,
    scratch_types=(
        [pltpu.VMEM((64, P1_W), jnp.float32) for _ in range(P1_NBUF)]
        + [pltpu.VMEM((P1_W // 2, 128), jnp.float32) for _ in range(P1_NBUF)]
        + [pltpu.SemaphoreType.DMA for _ in range(2 * P1_NBUF)]
    ),
    compiler_params=_tc_tiled,
)
def _repack_table(tt_hbm, tail_hbm, ts_hbm, *rest):
    ins = rest[:P1_NBUF]
    obs = rest[P1_NBUF : 2 * P1_NBUF]
    isem = rest[2 * P1_NBUF : 3 * P1_NBUF]
    osem = rest[3 * P1_NBUF :]

    w = _wid()
    nk = (NSTRIP - w + NW - 1) // NW
    riota = [lax.iota(jnp.int32, LANES) + 16 * j for j in range(4)]

    def strip_of(k):
        return w + k * NW

    def gather_in(b, k):
        c = strip_of(k)
        pltpu.async_copy(
            tt_hbm.at[:, pl.ds(c * P1_W, P1_W)], ins[b], isem[b]
        )

    for b in range(P1_NBUF):
        @pl.when(b < nk)
        def _prime(b=b):
            gather_in(b, b)

    def step(t, carry):
        for b in range(P1_NBUF):
            k = t * P1_NBUF + b

            @pl.when(k < nk)
            def _work(b=b, k=k):
                c = strip_of(k)
                pltpu.make_async_copy(
                    tt_hbm.at[:, pl.ds(c * P1_W, P1_W)], ins[b], isem[b]
                ).wait()

                @pl.when(k >= P1_NBUF)
                def _free_out():
                    pltpu.make_async_copy(
                        obs[b], ts_hbm.at[pl.ds(0, P1_W // 2)], osem[b]
                    ).wait()

                _transpose_strip(ins[b], obs[b], P1_W // 2, riota)
                pltpu.async_copy(
                    obs[b], ts_hbm.at[pl.ds(c * (P1_W // 2), P1_W // 2)], osem[b]
                )

                @pl.when(k + P1_NBUF < nk)
                def _refill():
                    gather_in(b, k + P1_NBUF)

        return carry

    lax.fori_loop(0, P1_NT, step, 0)

    for b in range(P1_NBUF):
        @pl.when(b < nk)
        def _drain(b=b):
            pltpu.make_async_copy(
                obs[b], ts_hbm.at[pl.ds(0, P1_W // 2)], osem[b]
            ).wait()

    @pl.when(w == NW - 1)
    def _tail():
        pltpu.sync_copy(tail_hbm, obs[0].at[pl.ds(0, 32)])
        pltpu.sync_copy(
            obs[0].at[pl.ds(0, 32)], ts_hbm.at[pl.ds(REM_BASE // 2, 32)]
        )


@functools.partial(
    pl.kernel,
    out_type=jax.ShapeDtypeStruct((TOK, D_MODEL, SEQ), jnp.float32),
    mesh=_mesh,
    scratch_types=(
        [pltpu.VMEM((TOK, SBLK), jnp.int32)]
        + [pltpu.VMEM((SBLK, 128), jnp.float32) for _ in range(P2_NBUF)]
        + [pltpu.VMEM((D_MODEL, SBLK), jnp.float32) for _ in range(P2_NOB)]
        + [pltpu.VMEM((SBLK,), jnp.int32) for _ in range(P2_NBUF)]
        + [pltpu.VMEM((SBLK,), jnp.int32) for _ in range(P2_NBUF)]
        + [pltpu.SemaphoreType.DMA for _ in range(P2_NBUF + P2_NOB)]
    ),
    compiler_params=_tc_tiled,
)
def _gather_emb(tokt_hbm, ts_hbm, out_hbm, idxslab, *rest):
    bufs = rest[:P2_NBUF]
    n0 = P2_NBUF
    obs = rest[n0 : n0 + P2_NOB]
    rowv = rest[n0 + P2_NOB : n0 + P2_NOB + P2_NBUF]
    parv = rest[n0 + P2_NOB + P2_NBUF : n0 + P2_NOB + 2 * P2_NBUF]
    gsem = rest[n0 + P2_NOB + 2 * P2_NBUF : 2 * n0 + P2_NOB + 2 * P2_NBUF]
    osem = rest[2 * n0 + P2_NOB + 2 * P2_NBUF :]

    w = _wid()
    s0 = w * SBLK

    pltpu.sync_copy(tokt_hbm.at[:, pl.ds(s0, SBLK)], idxslab)

    def prep_idx(b, t):
        for g in range(SBLK // LANES):
            tok = idxslab[t, pl.ds(g * LANES, LANES)]
            rowv[b][pl.ds(g * LANES, LANES)] = lax.shift_right_logical(tok, 1)
            parv[b][pl.ds(g * LANES, LANES)] = lax.shift_left(
                lax.bitwise_and(tok, 1), 6
            )

    def gather_start(b):
        pltpu.async_copy(ts_hbm.at[rowv[b]], bufs[b], gsem[b])

    for b in range(P2_NBUF):
        prep_idx(b, b)
        gather_start(b)

    riota128 = [
        (lax.iota(jnp.int32, LANES) + 16 * g) * 128 for g in range(SBLK // LANES)
    ]

    def extract(b, ob):
        b1 = bufs[b].reshape(1, SBLK * 128)
        zero = jnp.zeros((LANES,), jnp.int32)
        for g in range(SBLK // LANES):
            par128 = riota128[g] + parv[b][pl.ds(g * LANES, LANES)]

            @plsc.parallel_loop(0, D_MODEL, unroll=8)
            def drow(d, g=g, par128=par128):
                v = plsc.load_gather(b1, [zero, par128 + d])
                obs[ob][d, pl.ds(g * LANES, LANES)] = v

    def step(t2, carry):
        for b in range(P2_NBUF):
            t = t2 * P2_NBUF + b
            ob = b % P2_NOB
            pltpu.make_async_copy(ts_hbm.at[rowv[b]], bufs[b], gsem[b]).wait()

            @pl.when(t >= P2_NOB)
            def _free_out(ob=ob):
                pltpu.make_async_copy(
                    obs[ob], out_hbm.at[0, :, pl.ds(s0, SBLK)], osem[ob]
                ).wait()

            extract(b, ob)
            pltpu.async_copy(
                obs[ob], out_hbm.at[t, :, pl.ds(s0, SBLK)], osem[ob]
            )

            @pl.when(t + P2_NBUF < TOK)
            def _next(b=b, t=t):
                prep_idx(b, t + P2_NBUF)
                gather_start(b)

        return carry

    lax.fori_loop(0, TOK // P2_NBUF, step, 0)

    for ob in range(P2_NOB):
        pltpu.make_async_copy(
            obs[ob], out_hbm.at[0, :, pl.ds(s0, SBLK)], osem[ob]
        ).wait()


def kernel(tokens, table):
    tail = (table[REM_BASE:] * SCALE).reshape(32, 128)
    ts = _repack_table(table.T, tail)
    out3 = _gather_emb(tokens.astype(jnp.int32).T, ts)
    return out3.transpose(2, 0, 1)


# speedup vs baseline: 1.1824x; 1.1824x over previous
import functools

import jax
import jax.numpy as jnp
from jax import lax
from jax.experimental import pallas as pl
from jax.experimental.pallas import tpu as pltpu
from jax.experimental.pallas import tpu_sc as plsc

VOCAB = 1000000
D_MODEL = 64
SCALE = 8.0

NC, NS = 2, 16
NW = NC * NS
LANES = 16

P1_W = 256
NSTRIP = VOCAB // P1_W
REM_BASE = NSTRIP * P1_W
TSROWS = VOCAB // 2
P1_NBUF = 2
P1_MAXK = (NSTRIP + NW - 1) // NW
P1_NT = (P1_MAXK + P1_NBUF - 1) // P1_NBUF

SEQ, TOK = 4096, 200
SBLK = 128
P2_NBUF = 4
P2_NOB = 2

_mesh = plsc.VectorSubcoreMesh(
    core_axis_name="c", subcore_axis_name="s", num_cores=NC, num_subcores=NS
)
_tc_tiled = pltpu.CompilerParams(
    use_tc_tiling_on_sc=True, needs_layout_passes=False
)


def _wid():
    return lax.axis_index("s") * NC + lax.axis_index("c")


def _transpose_strip(in_v, ob_v, ncol, riota):
    iota = riota[0]

    @plsc.parallel_loop(0, ncol // LANES, unroll=2)
    def qblk(q):
        c0 = q * LANES
        for s in range(LANES):
            sig = lax.bitwise_and(iota + s, 15)
            cols = sig + c0
            orow = lax.shift_right_logical(sig, 1) + q * 8
            hcol = lax.shift_left(lax.bitwise_and(sig, 1), 6) + iota
            for a in range(4):
                v = plsc.load_gather(in_v, [riota[a], cols])
                plsc.store_scatter(ob_v, [orow, hcol + 16 * a], v * SCALE)


@functools.partial(
    pl.kernel,
    out_type=jax.ShapeDtypeStruct((TSROWS, 128), jnp.float32),
    mesh=_mesh,
    scratch_types=(
        [pltpu.VMEM((64, P1_W), jnp.float32) for _ in range(P1_NBUF)]
        + [pltpu.VMEM((P1_W // 2, 128), jnp.float32) for _ in range(P1_NBUF)]
        + [pltpu.SemaphoreType.DMA for _ in range(2 * P1_NBUF)]
    ),
    compiler_params=_tc_tiled,
)
def _repack_table(tt_hbm, tail_hbm, ts_hbm, *rest):
    ins = rest[:P1_NBUF]
    obs = rest[P1_NBUF : 2 * P1_NBUF]
    isem = rest[2 * P1_NBUF : 3 * P1_NBUF]
    osem = rest[3 * P1_NBUF :]

    w = _wid()
    nk = (NSTRIP - w + NW - 1) // NW
    riota = [lax.iota(jnp.int32, LANES) + 16 * j for j in range(4)]

    def strip_of(k):
        return w + k * NW

    def gather_in(b, k):
        c = strip_of(k)
        pltpu.async_copy(
            tt_hbm.at[:, pl.ds(c * P1_W, P1_W)], ins[b], isem[b]
        )

    for b in range(P1_NBUF):
        @pl.when(b < nk)
        def _prime(b=b):
            gather_in(b, b)

    def step(t, carry):
        for b in range(P1_NBUF):
            k = t * P1_NBUF + b

            @pl.when(k < nk)
            def _work(b=b, k=k):
                c = strip_of(k)
                pltpu.make_async_copy(
                    tt_hbm.at[:, pl.ds(c * P1_W, P1_W)], ins[b], isem[b]
                ).wait()

                @pl.when(k >= P1_NBUF)
                def _free_out():
                    pltpu.make_async_copy(
                        obs[b], ts_hbm.at[pl.ds(0, P1_W // 2)], osem[b]
                    ).wait()

                _transpose_strip(ins[b], obs[b], P1_W, riota)
                pltpu.async_copy(
                    obs[b], ts_hbm.at[pl.ds(c * (P1_W // 2), P1_W // 2)], osem[b]
                )

                @pl.when(k + P1_NBUF < nk)
                def _refill():
                    gather_in(b, k + P1_NBUF)

        return carry

    lax.fori_loop(0, P1_NT, step, 0)

    for b in range(P1_NBUF):
        @pl.when(b < nk)
        def _drain(b=b):
            pltpu.make_async_copy(
                obs[b], ts_hbm.at[pl.ds(0, P1_W // 2)], osem[b]
            ).wait()

    @pl.when(w == NW - 1)
    def _tail():
        pltpu.sync_copy(tail_hbm, obs[0].at[pl.ds(0, 32)])
        pltpu.sync_copy(
            obs[0].at[pl.ds(0, 32)], ts_hbm.at[pl.ds(REM_BASE // 2, 32)]
        )


@functools.partial(
    pl.kernel,
    out_type=jax.ShapeDtypeStruct((TOK, D_MODEL, SEQ), jnp.float32),
    mesh=_mesh,
    scratch_types=(
        [pltpu.VMEM((TOK, SBLK), jnp.int32)]
        + [pltpu.VMEM((SBLK, 128), jnp.float32) for _ in range(P2_NBUF)]
        + [pltpu.VMEM((D_MODEL, SBLK), jnp.float32) for _ in range(P2_NOB)]
        + [pltpu.VMEM((SBLK,), jnp.int32) for _ in range(P2_NBUF)]
        + [pltpu.VMEM((SBLK,), jnp.int32) for _ in range(P2_NBUF)]
        + [pltpu.SemaphoreType.DMA for _ in range(P2_NBUF + P2_NOB)]
    ),
    compiler_params=_tc_tiled,
)
def _gather_emb(tokt_hbm, ts_hbm, out_hbm, idxslab, *rest):
    bufs = rest[:P2_NBUF]
    n0 = P2_NBUF + P2_NOB
    obs = rest[P2_NBUF:n0]
    rowv = rest[n0 : n0 + P2_NBUF]
    parv = rest[n0 + P2_NBUF : n0 + 2 * P2_NBUF]
    gsem = rest[n0 + 2 * P2_NBUF : n0 + 3 * P2_NBUF]
    osem = rest[n0 + 3 * P2_NBUF :]

    w = _wid()
    s0 = w * SBLK

    pltpu.sync_copy(tokt_hbm.at[:, pl.ds(s0, SBLK)], idxslab)

    def prep_idx(b, t):
        for g in range(SBLK // LANES):
            tok = idxslab[t, pl.ds(g * LANES, LANES)]
            rowv[b][pl.ds(g * LANES, LANES)] = lax.shift_right_logical(tok, 1)
            parv[b][pl.ds(g * LANES, LANES)] = lax.shift_left(
                lax.bitwise_and(tok, 1), 6
            )

    def gather_start(b):
        pltpu.async_copy(ts_hbm.at[rowv[b]], bufs[b], gsem[b])

    for b in range(P2_NBUF):
        prep_idx(b, b)
        gather_start(b)

    iota = lax.iota(jnp.int32, LANES)
    riota = [iota + 16 * g for g in range(SBLK // LANES)]

    def extract(b, ob):
        for g in range(SBLK // LANES):
            par = parv[b][pl.ds(g * LANES, LANES)]

            @plsc.parallel_loop(0, LANES, unroll=4)
            def svec(s, g=g, par=par):
                dvs = lax.bitwise_and(iota + s, 15)
                for m in range(D_MODEL // LANES):
                    dv = dvs + m * LANES
                    v = plsc.load_gather(bufs[b], [riota[g], par + dv])
                    plsc.store_scatter(obs[ob], [dv, riota[g]], v)

    def step(t2, carry):
        for b in range(P2_NBUF):
            t = t2 * P2_NBUF + b
            ob = b % P2_NOB
            pltpu.make_async_copy(ts_hbm.at[rowv[b]], bufs[b], gsem[b]).wait()

            @pl.when(t >= P2_NOB)
            def _free_out(ob=ob):
                pltpu.make_async_copy(
                    obs[ob], out_hbm.at[0, :, pl.ds(s0, SBLK)], osem[ob]
                ).wait()

            extract(b, ob)
            pltpu.async_copy(
                obs[ob], out_hbm.at[t, :, pl.ds(s0, SBLK)], osem[ob]
            )

            @pl.when(t + P2_NBUF < TOK)
            def _next(b=b, t=t):
                prep_idx(b, t + P2_NBUF)
                gather_start(b)

        return carry

    lax.fori_loop(0, TOK // P2_NBUF, step, 0)

    for ob in range(P2_NOB):
        pltpu.make_async_copy(
            obs[ob], out_hbm.at[0, :, pl.ds(s0, SBLK)], osem[ob]
        ).wait()


def kernel(tokens, table):
    tail = (table[REM_BASE:] * SCALE).reshape(32, 128)
    ts = _repack_table(table.T, tail)
    out3 = _gather_emb(tokens.astype(jnp.int32).T, ts)
    return out3.transpose(2, 0, 1)


# speedup vs baseline: 2.9202x; 2.4697x over previous
import functools

import jax
import jax.numpy as jnp
from jax import lax
from jax.experimental import pallas as pl
from jax.experimental.pallas import tpu as pltpu
from jax.experimental.pallas import tpu_sc as plsc

VOCAB = 1000000
D_MODEL = 64
SCALE = 8.0

NC, NS = 2, 16
NW = NC * NS
LANES = 16

P1_W = 256
NSTRIP = VOCAB // P1_W
REM_BASE = NSTRIP * P1_W
TSROWS = VOCAB // 2
P1_NBUF = 2
P1_MAXK = (NSTRIP + NW - 1) // NW
P1_NT = (P1_MAXK + P1_NBUF - 1) // P1_NBUF

SEQ, TOK = 4096, 200
SBLK = 128
P2_NBUF = 4
P2_NOB = 2

_mesh = plsc.VectorSubcoreMesh(
    core_axis_name="c", subcore_axis_name="s", num_cores=NC, num_subcores=NS
)
_tc_tiled = pltpu.CompilerParams(
    use_tc_tiling_on_sc=True, needs_layout_passes=False
)


def _wid():
    return lax.axis_index("s") * NC + lax.axis_index("c")


def _transpose_strip(in_v, ob_v, ncol, riota):
    iota = riota[0]

    @plsc.parallel_loop(0, LANES, unroll=4)
    def sdiag(s):
        sig = lax.bitwise_and(iota + s, 15)
        orow0 = lax.shift_right_logical(sig, 1)
        hcols = [
            lax.shift_left(lax.bitwise_and(sig, 1), 6) + riota[a]
            for a in range(4)
        ]
        for q in range(ncol // LANES):
            cols = sig + q * LANES
            orow = orow0 + q * 8
            for a in range(4):
                v = plsc.load_gather(in_v, [riota[a], cols])
                plsc.store_scatter(ob_v, [orow, hcols[a]], v * SCALE)


@functools.partial(
    pl.kernel,
    out_type=jax.ShapeDtypeStruct((TSROWS, 128), jnp.float32),
    mesh=_mesh,
    scratch_types=(
        [pltpu.VMEM((64, P1_W), jnp.float32) for _ in range(P1_NBUF)]
        + [pltpu.VMEM((P1_W // 2, 128), jnp.float32) for _ in range(P1_NBUF)]
        + [pltpu.SemaphoreType.DMA for _ in range(2 * P1_NBUF)]
    ),
    compiler_params=_tc_tiled,
)
def _repack_table(tt_hbm, tail_hbm, ts_hbm, *rest):
    ins = rest[:P1_NBUF]
    obs = rest[P1_NBUF : 2 * P1_NBUF]
    isem = rest[2 * P1_NBUF : 3 * P1_NBUF]
    osem = rest[3 * P1_NBUF :]

    w = _wid()
    nk = (NSTRIP - w + NW - 1) // NW
    riota = [lax.iota(jnp.int32, LANES) + 16 * j for j in range(4)]

    def strip_of(k):
        return w + k * NW

    def gather_in(b, k):
        c = strip_of(k)
        pltpu.async_copy(
            tt_hbm.at[:, pl.ds(c * P1_W, P1_W)], ins[b], isem[b]
        )

    for b in range(P1_NBUF):
        @pl.when(b < nk)
        def _prime(b=b):
            gather_in(b, b)

    def step(t, carry):
        for b in range(P1_NBUF):
            k = t * P1_NBUF + b

            @pl.when(k < nk)
            def _work(b=b, k=k):
                c = strip_of(k)
                pltpu.make_async_copy(
                    tt_hbm.at[:, pl.ds(c * P1_W, P1_W)], ins[b], isem[b]
                ).wait()

                @pl.when(k >= P1_NBUF)
                def _free_out():
                    pltpu.make_async_copy(
                        obs[b], ts_hbm.at[pl.ds(0, P1_W // 2)], osem[b]
                    ).wait()

                _transpose_strip(ins[b], obs[b], P1_W, riota)
                pltpu.async_copy(
                    obs[b], ts_hbm.at[pl.ds(c * (P1_W // 2), P1_W // 2)], osem[b]
                )

                @pl.when(k + P1_NBUF < nk)
                def _refill():
                    gather_in(b, k + P1_NBUF)

        return carry

    lax.fori_loop(0, P1_NT, step, 0)

    for b in range(P1_NBUF):
        @pl.when(b < nk)
        def _drain(b=b):
            pltpu.make_async_copy(
                obs[b], ts_hbm.at[pl.ds(0, P1_W // 2)], osem[b]
            ).wait()

    @pl.when(w == NW - 1)
    def _tail():
        pltpu.sync_copy(tail_hbm, obs[0].at[pl.ds(0, 32)])
        pltpu.sync_copy(
            obs[0].at[pl.ds(0, 32)], ts_hbm.at[pl.ds(REM_BASE // 2, 32)]
        )


@functools.partial(
    pl.kernel,
    out_type=jax.ShapeDtypeStruct((TOK, D_MODEL, SEQ), jnp.float32),
    mesh=_mesh,
    scratch_types=(
        [pltpu.VMEM((TOK, SBLK), jnp.int32)]
        + [pltpu.VMEM((SBLK, 128), jnp.float32) for _ in range(P2_NBUF)]
        + [pltpu.VMEM((D_MODEL, SBLK), jnp.float32) for _ in range(P2_NOB)]
        + [pltpu.VMEM((SBLK,), jnp.int32) for _ in range(P2_NBUF)]
        + [pltpu.VMEM((SBLK,), jnp.int32) for _ in range(P2_NBUF)]
        + [pltpu.SemaphoreType.DMA for _ in range(P2_NBUF + P2_NOB)]
    ),
    compiler_params=_tc_tiled,
)
def _gather_emb(tokt_hbm, ts_hbm, out_hbm, idxslab, *rest):
    bufs = rest[:P2_NBUF]
    n0 = P2_NBUF + P2_NOB
    obs = rest[P2_NBUF:n0]
    rowv = rest[n0 : n0 + P2_NBUF]
    parv = rest[n0 + P2_NBUF : n0 + 2 * P2_NBUF]
    gsem = rest[n0 + 2 * P2_NBUF : n0 + 3 * P2_NBUF]
    osem = rest[n0 + 3 * P2_NBUF :]

    w = _wid()
    s0 = w * SBLK

    pltpu.sync_copy(tokt_hbm.at[:, pl.ds(s0, SBLK)], idxslab)

    def prep_idx(b, t):
        for g in range(SBLK // LANES):
            tok = idxslab[t, pl.ds(g * LANES, LANES)]
            rowv[b][pl.ds(g * LANES, LANES)] = lax.shift_right_logical(tok, 1)
            parv[b][pl.ds(g * LANES, LANES)] = lax.shift_left(
                lax.bitwise_and(tok, 1), 6
            )

    def gather_start(b):
        pltpu.async_copy(ts_hbm.at[rowv[b]], bufs[b], gsem[b])

    for b in range(P2_NBUF):
        prep_idx(b, b)
        gather_start(b)

    iota = lax.iota(jnp.int32, LANES)
    riota = [iota + 16 * g for g in range(SBLK // LANES)]

    def extract(b, ob):
        for g in range(SBLK // LANES):
            par = parv[b][pl.ds(g * LANES, LANES)]

            @plsc.parallel_loop(0, LANES, unroll=4)
            def svec(s, g=g, par=par):
                dvs = lax.bitwise_and(iota + s, 15)
                for m in range(D_MODEL // LANES):
                    dv = dvs + m * LANES
                    v = plsc.load_gather(bufs[b], [riota[g], par + dv])
                    plsc.store_scatter(obs[ob], [dv, riota[g]], v)

    def step(t2, carry):
        for b in range(P2_NBUF):
            t = t2 * P2_NBUF + b
            ob = b % P2_NOB
            pltpu.make_async_copy(ts_hbm.at[rowv[b]], bufs[b], gsem[b]).wait()

            @pl.when(t >= P2_NOB)
            def _free_out(ob=ob):
                pltpu.make_async_copy(
                    obs[ob], out_hbm.at[0, :, pl.ds(s0, SBLK)], osem[ob]
                ).wait()

            extract(b, ob)
            pltpu.async_copy(
                obs[ob], out_hbm.at[t, :, pl.ds(s0, SBLK)], osem[ob]
            )

            @pl.when(t + P2_NBUF < TOK)
            def _next(b=b, t=t):
                prep_idx(b, t + P2_NBUF)
                gather_start(b)

        return carry

    lax.fori_loop(0, TOK // P2_NBUF, step, 0)

    for ob in range(P2_NOB):
        pltpu.make_async_copy(
            obs[ob], out_hbm.at[0, :, pl.ds(s0, SBLK)], osem[ob]
        ).wait()


def kernel(tokens, table):
    tail = (table[REM_BASE:] * SCALE).reshape(32, 128)
    ts = _repack_table(table.T, tail)
    out3 = _gather_emb(tokens.astype(jnp.int32).T, ts)
    return out3.transpose(2, 0, 1)


# speedup vs baseline: 3.3204x; 1.1371x over previous
import functools

import jax
import jax.numpy as jnp
from jax import lax
from jax.experimental import pallas as pl
from jax.experimental.pallas import tpu as pltpu
from jax.experimental.pallas import tpu_sc as plsc

VOCAB = 1000000
D_MODEL = 64
SCALE = 8.0

NC, NS = 2, 16
NW = NC * NS
LANES = 16

P1_W = 256
NSTRIP = VOCAB // P1_W
REM_BASE = NSTRIP * P1_W
TSROWS = VOCAB // 2
P1_NBUF = 2
P1_MAXK = (NSTRIP + NW - 1) // NW
P1_NT = (P1_MAXK + P1_NBUF - 1) // P1_NBUF

SEQ, TOK = 4096, 200
SBLK = 128
P2_NBUF = 4
P2_NOB = 2

_mesh = plsc.VectorSubcoreMesh(
    core_axis_name="c", subcore_axis_name="s", num_cores=NC, num_subcores=NS
)
_tc_tiled = pltpu.CompilerParams(
    use_tc_tiling_on_sc=True, needs_layout_passes=False
)


def _wid():
    return lax.axis_index("s") * NC + lax.axis_index("c")


def _transpose_strip(in_v, ob_v, ncol, riota):
    iota = riota[0]

    @plsc.parallel_loop(0, LANES, unroll=4)
    def sdiag(s):
        sig = lax.bitwise_and(iota + s, 15)
        orow0 = lax.shift_right_logical(sig, 1)
        hcols = [
            lax.shift_left(lax.bitwise_and(sig, 1), 6) + riota[a]
            for a in range(4)
        ]
        for q in range(ncol // LANES):
            cols = sig + q * LANES
            orow = orow0 + q * 8
            for a in range(4):
                v = plsc.load_gather(in_v, [riota[a], cols])
                plsc.store_scatter(ob_v, [orow, hcols[a]], v * SCALE)


@functools.partial(
    pl.kernel,
    out_type=jax.ShapeDtypeStruct((TSROWS, 128), jnp.float32),
    mesh=_mesh,
    scratch_types=(
        [pltpu.VMEM((64, P1_W), jnp.float32) for _ in range(P1_NBUF)]
        + [pltpu.VMEM((P1_W // 2, 128), jnp.float32) for _ in range(P1_NBUF)]
        + [pltpu.SemaphoreType.DMA for _ in range(2 * P1_NBUF)]
    ),
    compiler_params=_tc_tiled,
)
def _repack_table(tt_hbm, tail_hbm, ts_hbm, *rest):
    ins = rest[:P1_NBUF]
    obs = rest[P1_NBUF : 2 * P1_NBUF]
    isem = rest[2 * P1_NBUF : 3 * P1_NBUF]
    osem = rest[3 * P1_NBUF :]

    w = _wid()
    nk = (NSTRIP - w + NW - 1) // NW
    riota = [lax.iota(jnp.int32, LANES) + 16 * j for j in range(4)]

    def strip_of(k):
        return w + k * NW

    def gather_in(b, k):
        c = strip_of(k)
        pltpu.async_copy(
            tt_hbm.at[:, pl.ds(c * P1_W, P1_W)], ins[b], isem[b]
        )

    for b in range(P1_NBUF):
        @pl.when(b < nk)
        def _prime(b=b):
            gather_in(b, b)

    def step(t, carry):
        for b in range(P1_NBUF):
            k = t * P1_NBUF + b

            @pl.when(k < nk)
            def _work(b=b, k=k):
                c = strip_of(k)
                pltpu.make_async_copy(
                    tt_hbm.at[:, pl.ds(c * P1_W, P1_W)], ins[b], isem[b]
                ).wait()

                @pl.when(k >= P1_NBUF)
                def _free_out():
                    pltpu.make_async_copy(
                        obs[b], ts_hbm.at[pl.ds(0, P1_W // 2)], osem[b]
                    ).wait()

                _transpose_strip(ins[b], obs[b], P1_W, riota)
                pltpu.async_copy(
                    obs[b], ts_hbm.at[pl.ds(c * (P1_W // 2), P1_W // 2)], osem[b]
                )

                @pl.when(k + P1_NBUF < nk)
                def _refill():
                    gather_in(b, k + P1_NBUF)

        return carry

    lax.fori_loop(0, P1_NT, step, 0)

    for b in range(P1_NBUF):
        @pl.when(b < nk)
        def _drain(b=b):
            pltpu.make_async_copy(
                obs[b], ts_hbm.at[pl.ds(0, P1_W // 2)], osem[b]
            ).wait()

    @pl.when(w == NW - 1)
    def _tail():
        pltpu.sync_copy(tail_hbm, obs[0].at[pl.ds(0, 32)])
        pltpu.sync_copy(
            obs[0].at[pl.ds(0, 32)], ts_hbm.at[pl.ds(REM_BASE // 2, 32)]
        )


@functools.partial(
    pl.kernel,
    out_type=jax.ShapeDtypeStruct((TOK, D_MODEL, SEQ), jnp.float32),
    mesh=_mesh,
    scratch_types=(
        [pltpu.VMEM((TOK, SBLK), jnp.int32)]
        + [pltpu.VMEM((SBLK, 128), jnp.float32) for _ in range(P2_NBUF)]
        + [pltpu.VMEM((D_MODEL, SBLK), jnp.float32) for _ in range(P2_NOB)]
        + [pltpu.VMEM((SBLK,), jnp.int32) for _ in range(P2_NBUF)]
        + [pltpu.VMEM((SBLK,), jnp.int32) for _ in range(P2_NBUF)]
        + [pltpu.SemaphoreType.DMA for _ in range(P2_NBUF + P2_NOB)]
    ),
    compiler_params=_tc_tiled,
)
def _gather_emb(tokt_hbm, ts_hbm, out_hbm, idxslab, *rest):
    bufs = rest[:P2_NBUF]
    n0 = P2_NBUF + P2_NOB
    obs = rest[P2_NBUF:n0]
    rowv = rest[n0 : n0 + P2_NBUF]
    parv = rest[n0 + P2_NBUF : n0 + 2 * P2_NBUF]
    gsem = rest[n0 + 2 * P2_NBUF : n0 + 3 * P2_NBUF]
    osem = rest[n0 + 3 * P2_NBUF :]

    w = _wid()
    s0 = w * SBLK

    pltpu.sync_copy(tokt_hbm.at[:, pl.ds(s0, SBLK)], idxslab)

    def prep_idx(b, t):
        for g in range(SBLK // LANES):
            tok = idxslab[t, pl.ds(g * LANES, LANES)]
            rowv[b][pl.ds(g * LANES, LANES)] = lax.shift_right_logical(tok, 1)
            parv[b][pl.ds(g * LANES, LANES)] = lax.shift_left(
                lax.bitwise_and(tok, 1), 6
            )

    def gather_start(b):
        pltpu.async_copy(ts_hbm.at[rowv[b]], bufs[b], gsem[b])

    for b in range(P2_NBUF):
        prep_idx(b, b)
        gather_start(b)

    iota = lax.iota(jnp.int32, LANES)
    riota = [iota + 16 * g for g in range(SBLK // LANES)]

    def extract(b, ob):
        for g in range(SBLK // LANES):
            par = parv[b][pl.ds(g * LANES, LANES)]

            @plsc.parallel_loop(0, LANES, unroll=8)
            def svec(s, g=g, par=par):
                dvs = lax.bitwise_and(iota + s, 15)
                pdv = par + dvs
                for m in range(D_MODEL // LANES):
                    v = plsc.load_gather(bufs[b], [riota[g], pdv + m * LANES])
                    plsc.store_scatter(obs[ob], [dvs + m * LANES, riota[g]], v)

    def step(t2, carry):
        for b in range(P2_NBUF):
            t = t2 * P2_NBUF + b
            ob = b % P2_NOB
            pltpu.make_async_copy(ts_hbm.at[rowv[b]], bufs[b], gsem[b]).wait()

            @pl.when(t >= P2_NOB)
            def _free_out(ob=ob):
                pltpu.make_async_copy(
                    obs[ob], out_hbm.at[0, :, pl.ds(s0, SBLK)], osem[ob]
                ).wait()

            extract(b, ob)
            pltpu.async_copy(
                obs[ob], out_hbm.at[t, :, pl.ds(s0, SBLK)], osem[ob]
            )

            @pl.when(t + P2_NBUF < TOK)
            def _next(b=b, t=t):
                prep_idx(b, t + P2_NBUF)
                gather_start(b)

        return carry

    lax.fori_loop(0, TOK // P2_NBUF, step, 0)

    for ob in range(P2_NOB):
        pltpu.make_async_copy(
            obs[ob], out_hbm.at[0, :, pl.ds(s0, SBLK)], osem[ob]
        ).wait()


def kernel(tokens, table):
    tail = (table[REM_BASE:] * SCALE).reshape(32, 128)
    ts = _repack_table(table.T, tail)
    out3 = _gather_emb(tokens.astype(jnp.int32).T, ts)
    return out3.transpose(2, 0, 1)
